# baseline (speedup 1.0000x reference)
import jax
import jax.numpy as jnp
from jax.experimental import pallas as pl
from jax.experimental.pallas import tpu as pltpu

EPS = 1e-5
NB = 8
Q = 2
C2 = 64
T = 2048
DH = 32


def _mm(a, b):
    return jnp.dot(a.astype(jnp.bfloat16), b,
                   preferred_element_type=jnp.float32)


def _blockdiag(M, q):
    a, b = M.shape
    z = jnp.zeros((a, b), M.dtype)
    rows = [jnp.concatenate([M if j == i else z for j in range(q)], axis=1)
            for i in range(q)]
    return jnp.concatenate(rows, axis=0)


def _block_kernel(x_ref, g0_ref, b0_ref, Wsk_ref, bsk_ref,
                  Wsb_ref, bsb_ref, Wsc_ref, bsc_ref, g1_ref, b1_ref,
                  Wse_ref, bse_ref, Wtb_ref, btb_ref, Wtc_ref, btc_ref,
                  Wte_ref, bte_ref, out_ref,
                  a0f, c0f, bskf, Ws1, cs1, Wseq, bseq, Wt1, ct1, Wteq, bteq):
    s = 1.0 / (1.0 + EPS) ** 0.5
    bf = jnp.bfloat16

    @pl.when(pl.program_id(0) == 0)
    def _prep():
        a0f[...] = jnp.broadcast_to(g0_ref[...] * s, (C2, T))
        c0f[...] = jnp.broadcast_to(b0_ref[...], (C2, T))
        bskf[...] = jnp.broadcast_to(bsk_ref[...], (C2, T))
        gs = g1_ref[...] * s
        Ws1[...] = _blockdiag(jnp.dot(Wsb_ref[...], Wsc_ref[...]) * gs,
                              Q).astype(bf)
        cs1[...] = jnp.tile(jnp.dot(bsb_ref[...], Wsc_ref[...])
                            + bsc_ref[...] * 1.0, (1, Q)) * jnp.tile(
                                gs, (1, Q)) + jnp.tile(b1_ref[...], (1, Q))
        Wt1[...] = _blockdiag(jnp.dot(Wtb_ref[...], Wtc_ref[...]) * gs,
                              Q).astype(bf)
        ct1[...] = jnp.tile(jnp.dot(btb_ref[...], Wtc_ref[...])
                            + btc_ref[...] * 1.0, (1, Q)) * jnp.tile(
                                gs, (1, Q)) + jnp.tile(b1_ref[...], (1, Q))
        Wseq[...] = _blockdiag(Wse_ref[...], Q).astype(bf)
        bseq[...] = jnp.tile(bse_ref[...], (1, Q))
        Wteq[...] = _blockdiag(Wte_ref[...], Q).astype(bf)
        bteq[...] = jnp.tile(bte_ref[...], (1, Q))

    X = x_ref[...]
    xb = X * a0f[...][None] + c0f[...][None]
    xbh = xb.astype(bf)
    bskv = bskf[...]
    W0 = Wsk_ref[0].astype(bf)
    W1 = Wsk_ref[1].astype(bf)
    W2 = Wsk_ref[2].astype(bf)

    res_parts = []
    z = jnp.zeros((C2, 1), bf)
    for b in range(NB):
        sl = xbh[b]
        xm1 = jnp.concatenate([z, sl[:, :-1]], axis=1)
        xp1 = jnp.concatenate([sl[:, 1:], z], axis=1)
        r = (jnp.dot(W0, xm1, preferred_element_type=jnp.float32)
             + jnp.dot(W1, sl, preferred_element_type=jnp.float32)
             + jnp.dot(W2, xp1, preferred_element_type=jnp.float32))
        res_parts.append(jax.nn.relu(r + bskv))
    res = jnp.concatenate(res_parts, axis=0)

    xs = xbh.reshape(NB * C2, T)
    cs1v = cs1[...][0]
    bsev = bseq[...][0]
    ct1v = ct1[...][0]
    btev = bteq[...][0]
    Wc = 64 * Q
    n = T // Wc
    sls = [slice(Wc * i, Wc * (i + 1)) for i in range(n)]
    U = [jax.nn.relu(jnp.dot(xs[:, sl], Ws1[...],
                             preferred_element_type=jnp.float32) + cs1v)
         for sl in sls]
    H = [res[:, sls[i]] + (_mm(U[i], Wseq[...]) + bsev) for i in range(n)]
    V = [jax.nn.relu(_mm(h, Wt1[...]) + ct1v) for h in H]
    for i in range(n):
        o = res[:, sls[i]] + (_mm(V[i], Wteq[...]) + btev)
        out_ref[:, :, sls[i]] = o.reshape(NB, C2, Wc)


def kernel(x, g0, b0, Wskip, bskip, Wsb, bsb, Wsc, bsc, g1, b1, Wse, bse,
           Wtb, btb, Wtc, btc, Wte, bte, edge_index, train):
    B = x.shape[0]
    dh = DH
    bf = jnp.bfloat16

    Wsk = jnp.moveaxis(Wskip, 2, 0)

    full = lambda shp: pl.BlockSpec(shp, lambda b: (0,) * len(shp))
    return pl.pallas_call(
        _block_kernel,
        grid=(B // NB,),
        in_specs=[
            pl.BlockSpec((NB, C2, T), lambda b: (b, 0, 0)),
            full((C2, 1)), full((C2, 1)),
            full((3, 2 * dh, C2)), full((2 * dh, 1)),
            full((C2, dh)), full((1, dh)),
            full((dh, dh)), full((1, dh)),
            full((1, dh)), full((1, dh)),
            full((dh, 2 * dh)), full((1, 2 * dh)),
            full((2 * dh, dh)), full((1, dh)),
            full((dh, dh)), full((1, dh)),
            full((dh, 2 * dh)), full((1, 2 * dh)),
        ],
        out_specs=pl.BlockSpec((NB, C2, T), lambda b: (b, 0, 0)),
        out_shape=jax.ShapeDtypeStruct((B, C2, T), jnp.float32),
        scratch_shapes=[
            pltpu.VMEM((C2, T), jnp.float32),
            pltpu.VMEM((C2, T), jnp.float32),
            pltpu.VMEM((C2, T), jnp.float32),
            pltpu.VMEM((C2 * Q, dh * Q), bf),
            pltpu.VMEM((1, dh * Q), jnp.float32),
            pltpu.VMEM((dh * Q, 2 * dh * Q), bf),
            pltpu.VMEM((1, 2 * dh * Q), jnp.float32),
            pltpu.VMEM((C2 * Q, dh * Q), bf),
            pltpu.VMEM((1, dh * Q), jnp.float32),
            pltpu.VMEM((dh * Q, 2 * dh * Q), bf),
            pltpu.VMEM((1, 2 * dh * Q), jnp.float32),
        ],
    )(x, g0.reshape(C2, 1), b0.reshape(C2, 1), Wsk,
      bskip.reshape(2 * dh, 1), Wsb, bsb.reshape(1, dh), Wsc,
      bsc.reshape(1, dh), g1.reshape(1, dh), b1.reshape(1, dh), Wse,
      bse.reshape(1, 2 * dh), Wtb, btb.reshape(1, dh), Wtc,
      btc.reshape(1, dh), Wte, bte.reshape(1, 2 * dh))


# speedup vs baseline: 1.0170x; 1.0170x over previous
import jax
import jax.numpy as jnp
from jax.experimental import pallas as pl
from jax.experimental.pallas import tpu as pltpu

EPS = 1e-5
NB = 8
Q = 2
C2 = 64
T = 2048
DH = 32


def _mm(a, b):
    return jnp.dot(a.astype(jnp.bfloat16), b,
                   preferred_element_type=jnp.float32)


def _blockdiag(M, q):
    a, b = M.shape
    z = jnp.zeros((a, b), M.dtype)
    rows = [jnp.concatenate([M if j == i else z for j in range(q)], axis=1)
            for i in range(q)]
    return jnp.concatenate(rows, axis=0)


def _block_kernel(x_ref, g0_ref, b0_ref, Wsk_ref, bsk_ref,
                  Wsb_ref, bsb_ref, Wsc_ref, bsc_ref, g1_ref, b1_ref,
                  Wse_ref, bse_ref, Wtb_ref, btb_ref, Wtc_ref, btc_ref,
                  Wte_ref, bte_ref, out_ref,
                  a0f, c0f, bskf, Ws1, cs1, Wseq, bseq, Wt1, ct1, Wteq, bteq,
                  Wcat):
    s = 1.0 / (1.0 + EPS) ** 0.5
    bf = jnp.bfloat16

    @pl.when(pl.program_id(0) == 0)
    def _prep():
        a0f[...] = jnp.broadcast_to(g0_ref[...] * s, (C2, T))
        c0f[...] = jnp.broadcast_to(b0_ref[...], (C2, T))
        bskf[...] = jnp.broadcast_to(bsk_ref[...], (C2, T))
        gs = g1_ref[...] * s
        Ws1[...] = _blockdiag(jnp.dot(Wsb_ref[...], Wsc_ref[...]) * gs,
                              Q).astype(bf)
        cs1[...] = jnp.tile(jnp.dot(bsb_ref[...], Wsc_ref[...])
                            + bsc_ref[...] * 1.0, (1, Q)) * jnp.tile(
                                gs, (1, Q)) + jnp.tile(b1_ref[...], (1, Q))
        Wt1[...] = _blockdiag(jnp.dot(Wtb_ref[...], Wtc_ref[...]) * gs,
                              Q).astype(bf)
        ct1[...] = jnp.tile(jnp.dot(btb_ref[...], Wtc_ref[...])
                            + btc_ref[...] * 1.0, (1, Q)) * jnp.tile(
                                gs, (1, Q)) + jnp.tile(b1_ref[...], (1, Q))
        Wseq[...] = _blockdiag(Wse_ref[...], Q).astype(bf)
        bseq[...] = jnp.tile(bse_ref[...], (1, Q))
        Wteq[...] = _blockdiag(Wte_ref[...], Q).astype(bf)
        bteq[...] = jnp.tile(bte_ref[...], (1, Q))
        Wcat[...] = jnp.concatenate(
            [Wsk_ref[0], Wsk_ref[1], Wsk_ref[2]], axis=1).astype(bf)

    X = x_ref[...]
    xb = X * a0f[...][None] + c0f[...][None]
    xbh = xb.astype(bf)
    bskv = bskf[...]
    Wc3 = Wcat[...]

    res_parts = []
    z = jnp.zeros((C2, 1), bf)
    for b in range(NB):
        sl = xbh[b]
        xm1 = jnp.concatenate([z, sl[:, :-1]], axis=1)
        xp1 = jnp.concatenate([sl[:, 1:], z], axis=1)
        s3 = jnp.concatenate([xm1, sl, xp1], axis=0)
        r = jnp.dot(Wc3, s3, preferred_element_type=jnp.float32)
        res_parts.append(jax.nn.relu(r + bskv))
    res = jnp.concatenate(res_parts, axis=0)

    xs = xbh.reshape(NB * C2, T)
    cs1v = cs1[...][0]
    bsev = bseq[...][0]
    ct1v = ct1[...][0]
    btev = bteq[...][0]
    Wc = 64 * Q
    n = T // Wc
    sls = [slice(Wc * i, Wc * (i + 1)) for i in range(n)]
    U = [jax.nn.relu(jnp.dot(xs[:, sl], Ws1[...],
                             preferred_element_type=jnp.float32) + cs1v)
         for sl in sls]
    H = [res[:, sls[i]] + (_mm(U[i], Wseq[...]) + bsev) for i in range(n)]
    V = [jax.nn.relu(_mm(h, Wt1[...]) + ct1v) for h in H]
    for i in range(n):
        o = res[:, sls[i]] + (_mm(V[i], Wteq[...]) + btev)
        out_ref[:, :, sls[i]] = o.reshape(NB, C2, Wc)


def kernel(x, g0, b0, Wskip, bskip, Wsb, bsb, Wsc, bsc, g1, b1, Wse, bse,
           Wtb, btb, Wtc, btc, Wte, bte, edge_index, train):
    B = x.shape[0]
    dh = DH
    bf = jnp.bfloat16

    Wsk = jnp.moveaxis(Wskip, 2, 0)

    full = lambda shp: pl.BlockSpec(shp, lambda b: (0,) * len(shp))
    return pl.pallas_call(
        _block_kernel,
        grid=(B // NB,),
        in_specs=[
            pl.BlockSpec((NB, C2, T), lambda b: (b, 0, 0)),
            full((C2, 1)), full((C2, 1)),
            full((3, 2 * dh, C2)), full((2 * dh, 1)),
            full((C2, dh)), full((1, dh)),
            full((dh, dh)), full((1, dh)),
            full((1, dh)), full((1, dh)),
            full((dh, 2 * dh)), full((1, 2 * dh)),
            full((2 * dh, dh)), full((1, dh)),
            full((dh, dh)), full((1, dh)),
            full((dh, 2 * dh)), full((1, 2 * dh)),
        ],
        out_specs=pl.BlockSpec((NB, C2, T), lambda b: (b, 0, 0)),
        out_shape=jax.ShapeDtypeStruct((B, C2, T), jnp.float32),
        scratch_shapes=[
            pltpu.VMEM((C2, T), jnp.float32),
            pltpu.VMEM((C2, T), jnp.float32),
            pltpu.VMEM((C2, T), jnp.float32),
            pltpu.VMEM((C2 * Q, dh * Q), bf),
            pltpu.VMEM((1, dh * Q), jnp.float32),
            pltpu.VMEM((dh * Q, 2 * dh * Q), bf),
            pltpu.VMEM((1, 2 * dh * Q), jnp.float32),
            pltpu.VMEM((C2 * Q, dh * Q), bf),
            pltpu.VMEM((1, dh * Q), jnp.float32),
            pltpu.VMEM((dh * Q, 2 * dh * Q), bf),
            pltpu.VMEM((1, 2 * dh * Q), jnp.float32),
            pltpu.VMEM((C2, 3 * C2), jnp.bfloat16),
        ],
    )(x, g0.reshape(C2, 1), b0.reshape(C2, 1), Wsk,
      bskip.reshape(2 * dh, 1), Wsb, bsb.reshape(1, dh), Wsc,
      bsc.reshape(1, dh), g1.reshape(1, dh), b1.reshape(1, dh), Wse,
      bse.reshape(1, 2 * dh), Wtb, btb.reshape(1, dh), Wtc,
      btc.reshape(1, dh), Wte, bte.reshape(1, 2 * dh))
